# baseline (speedup 1.0000x reference)
import functools

import jax
import jax.numpy as jnp
from jax import lax
from jax.experimental import pallas as pl
from jax.experimental.pallas import tpu as pltpu
from jax.experimental.pallas import tpu_sc as plsc

N = 10000
NPAD = 10240
E = 320000
NC = 2
NS = 16
NW = NC * NS
K = 128
EPB = 2 * (-(-E // (NW * K * 2)))
NBLK = NW * EPB
BPT = NBLK // NS
EPAD = NBLK * K
RPT = NPAD // NS
ZR = 64
RB = 640
F32 = jnp.float32


def _make_sc_agg(d2):
    mesh = plsc.VectorSubcoreMesh(core_axis_name="c", subcore_axis_name="s")
    scratch = [
        pltpu.VMEM((2, K), jnp.int32),
        pltpu.VMEM((2, K), jnp.int32),
        pltpu.VMEM((2, K, d2), F32),
        pltpu.VMEM((ZR, d2), F32),
        pltpu.VMEM_SHARED((NPAD, d2), F32),
        pltpu.VMEM_SHARED((NPAD, d2), F32),
    ] + [pltpu.SemaphoreType.DMA] * 8

    @functools.partial(
        pl.kernel,
        out_type=jax.ShapeDtypeStruct((NC, NPAD, d2), F32),
        mesh=mesh,
        scratch_types=scratch,
        compiler_params=pltpu.CompilerParams(use_tc_tiling_on_sc=False),
    )
    def body(yl_hbm, yr_hbm, src_hbm, dst_hbm, out_hbm, sidx, didx, rows,
             zbuf, yloc, acc, sf0, sf1, sd0, sd1, sg0, sg1, ss0, ss1):
        s_src, s_dst = (sf0, sf1), (sd0, sd1)
        s_g, s_s = (sg0, sg1), (ss0, ss1)
        c = lax.axis_index("c")
        s = lax.axis_index("s")

        fill = jnp.zeros((16,), F32)

        @pl.loop(0, ZR)
        def _(i):
            @pl.loop(0, d2 // 16)
            def _(j):
                zbuf[i, pl.ds(j * 16, 16)] = fill

        @pl.loop(0, RPT // ZR)
        def _(t):
            pltpu.sync_copy(zbuf, acc.at[pl.ds(s * RPT + t * ZR, ZR)])

        @pl.when(c == 0)
        def _():
            pltpu.sync_copy(yl_hbm.at[pl.ds(s * RPT, RPT)],
                            yloc.at[pl.ds(s * RPT, RPT)])

        @pl.when(c == 1)
        def _():
            pltpu.sync_copy(yr_hbm.at[pl.ds(s * RPT, RPT)],
                            yloc.at[pl.ds(s * RPT, RPT)])

        plsc.subcore_barrier()

        base0 = s * (BPT * K)

        def fetch(t, p):
            off = base0 + t * K
            pltpu.async_copy(src_hbm.at[pl.ds(off, K)], sidx.at[p], s_src[p])
            pltpu.async_copy(dst_hbm.at[pl.ds(off, K)], didx.at[p], s_dst[p])

        def wait_fetch(p):
            pltpu.make_async_copy(src_hbm.at[pl.ds(0, K)], sidx.at[p],
                                  s_src[p]).wait()
            pltpu.make_async_copy(dst_hbm.at[pl.ds(0, K)], didx.at[p],
                                  s_dst[p]).wait()

        def do_gather(p):
            pltpu.async_copy(yloc.at[sidx.at[p]], rows.at[p], s_g[p])

        def wait_gather(p):
            pltpu.make_async_copy(yloc.at[sidx.at[p]], rows.at[p],
                                  s_g[p]).wait()

        def do_scatter(p):
            pltpu.async_copy(rows.at[p], acc.at[didx.at[p]], s_s[p], add=True)

        def wait_scatter(p):
            pltpu.make_async_copy(rows.at[p], acc.at[didx.at[p]],
                                  s_s[p]).wait()

        fetch(0, 0)
        fetch(1, 1)
        wait_fetch(0)
        do_gather(0)

        @pl.loop(0, (BPT - 2) // 2)
        def _(it):
            t0 = it * 2
            for p in (0, 1):
                wait_gather(p)
                do_scatter(p)
                wait_fetch(1 - p)
                do_gather(1 - p)
                wait_scatter(p)
                fetch(t0 + p + 2, p)

        wait_gather(0)
        do_scatter(0)
        wait_fetch(1)
        do_gather(1)
        wait_scatter(0)
        wait_gather(1)
        do_scatter(1)
        wait_scatter(1)

        plsc.subcore_barrier()
        pltpu.sync_copy(acc.at[pl.ds(s * RPT, RPT)],
                        out_hbm.at[c, pl.ds(s * RPT, RPT)])

    return body


def _make_sc_count(d):
    mesh = plsc.VectorSubcoreMesh(core_axis_name="c", subcore_axis_name="s")
    scratch = [
        pltpu.VMEM((2, K), jnp.int32),
        pltpu.VMEM((K, d), F32),
        pltpu.VMEM((ZR, d), F32),
        pltpu.VMEM_SHARED((NPAD, d), F32),
    ] + [pltpu.SemaphoreType.DMA] * 4

    @functools.partial(
        pl.kernel,
        out_type=jax.ShapeDtypeStruct((NC, NPAD, d), F32),
        mesh=mesh,
        scratch_types=scratch,
        compiler_params=pltpu.CompilerParams(use_tc_tiling_on_sc=False),
    )
    def body(dst_hbm, out_hbm, didx, rows, zbuf, acc, sd0, sd1, ss0, ss1):
        s_dst, s_s = (sd0, sd1), (ss0, ss1)
        c = lax.axis_index("c")
        s = lax.axis_index("s")
        wid = s * NC + c
        base0 = wid * (EPB * K)

        fill = jnp.zeros((16,), F32)
        ones = jnp.ones((16,), F32)

        @pl.loop(0, ZR)
        def _(i):
            @pl.loop(0, d // 16)
            def _(j):
                zbuf[i, pl.ds(j * 16, 16)] = fill

        @pl.loop(0, K)
        def _(i):
            @pl.loop(0, d // 16)
            def _(j):
                rows[i, pl.ds(j * 16, 16)] = ones

        @pl.loop(0, RPT // ZR)
        def _(t):
            pltpu.sync_copy(zbuf, acc.at[pl.ds(s * RPT + t * ZR, ZR)])

        plsc.subcore_barrier()

        def fetch(t, p):
            off = base0 + t * K
            pltpu.async_copy(dst_hbm.at[pl.ds(off, K)], didx.at[p], s_dst[p])

        def wait_fetch(p):
            pltpu.make_async_copy(dst_hbm.at[pl.ds(0, K)], didx.at[p],
                                  s_dst[p]).wait()

        def do_scatter(p):
            pltpu.async_copy(rows, acc.at[didx.at[p]], s_s[p], add=True)

        def wait_scatter(p):
            pltpu.make_async_copy(rows, acc.at[didx.at[p]], s_s[p]).wait()

        fetch(0, 0)
        fetch(1, 1)
        wait_fetch(0)
        do_scatter(0)

        @pl.loop(0, (EPB - 2) // 2)
        def _(it):
            t0 = it * 2
            for p in (0, 1):
                wait_fetch(1 - p)
                do_scatter(1 - p)
                wait_scatter(p)
                fetch(t0 + p + 2, p)

        wait_fetch(1)
        do_scatter(1)
        wait_scatter(0)
        wait_scatter(1)

        plsc.subcore_barrier()
        pltpu.sync_copy(acc.at[pl.ds(s * RPT, RPT)],
                        out_hbm.at[c, pl.ds(s * RPT, RPT)])

    return body


_sc_count = _make_sc_count(16)
_sc_agg = {64: _make_sc_agg(64), 32: _make_sc_agg(32)}


def _dis_y_body(d0, d1, x, w, dis_o, yl_o, yr_o):
    deg = d0[...] + d1[...] + 1.0
    dis = lax.rsqrt(deg)
    dis_o[...] = dis
    y = dis * jnp.dot(x[...], w[...], preferred_element_type=F32)
    d2 = y.shape[1] // 2
    yl_o[...] = y[:, :d2]
    yr_o[...] = y[:, d2:]


def _mid_body(p0, p1, yl, yr, dis, b, w, yl_o, yr_o):
    h = dis[...] * jnp.concatenate(
        [p0[...] + yl[...], p1[...] + yr[...]], axis=1) + b[...]
    h = jnp.maximum(h, 0.0)
    y = dis[...] * jnp.dot(h, w[...], preferred_element_type=F32)
    d2 = y.shape[1] // 2
    yl_o[...] = y[:, :d2]
    yr_o[...] = y[:, d2:]


def _fin_body(p0, p1, yl, yr, dis, b, out):
    out[...] = dis[...] * jnp.concatenate(
        [p0[...] + yl[...], p1[...] + yr[...]], axis=1) + b[...]


def _col(i):
    return (i, 0)


def _rep(i):
    return (0, 0)


def _tc_dis_y(dp0, dp1, xp, w):
    din, dout = w.shape
    d2 = dout // 2
    return pl.pallas_call(
        _dis_y_body,
        grid=(NPAD // RB,),
        in_specs=[
            pl.BlockSpec((RB, 1), _col),
            pl.BlockSpec((RB, 1), _col),
            pl.BlockSpec((RB, din), _col),
            pl.BlockSpec((din, dout), _rep),
        ],
        out_specs=[
            pl.BlockSpec((RB, 1), _col),
            pl.BlockSpec((RB, d2), _col),
            pl.BlockSpec((RB, d2), _col),
        ],
        out_shape=[
            jax.ShapeDtypeStruct((NPAD, 1), F32),
            jax.ShapeDtypeStruct((NPAD, d2), F32),
            jax.ShapeDtypeStruct((NPAD, d2), F32),
        ],
    )(dp0, dp1, xp, w)


def _tc_mid(p0, p1, yl, yr, dis, b, w):
    din, dout = w.shape
    d2i, d2o = din // 2, dout // 2
    return pl.pallas_call(
        _mid_body,
        grid=(NPAD // RB,),
        in_specs=[
            pl.BlockSpec((RB, d2i), _col),
            pl.BlockSpec((RB, d2i), _col),
            pl.BlockSpec((RB, d2i), _col),
            pl.BlockSpec((RB, d2i), _col),
            pl.BlockSpec((RB, 1), _col),
            pl.BlockSpec((1, din), _rep),
            pl.BlockSpec((din, dout), _rep),
        ],
        out_specs=[
            pl.BlockSpec((RB, d2o), _col),
            pl.BlockSpec((RB, d2o), _col),
        ],
        out_shape=[
            jax.ShapeDtypeStruct((NPAD, d2o), F32),
            jax.ShapeDtypeStruct((NPAD, d2o), F32),
        ],
    )(p0, p1, yl, yr, dis, b, w)


def _tc_fin(p0, p1, yl, yr, dis, b):
    d2 = p0.shape[1]
    return pl.pallas_call(
        _fin_body,
        grid=(NPAD // RB,),
        in_specs=[
            pl.BlockSpec((RB, d2), _col),
            pl.BlockSpec((RB, d2), _col),
            pl.BlockSpec((RB, d2), _col),
            pl.BlockSpec((RB, d2), _col),
            pl.BlockSpec((RB, 1), _col),
            pl.BlockSpec((1, 2 * d2), _rep),
        ],
        out_specs=pl.BlockSpec((RB, 2 * d2), _col),
        out_shape=jax.ShapeDtypeStruct((NPAD, 2 * d2), F32),
    )(p0, p1, yl, yr, dis, b)


def kernel(x, edge_index, W1, b1, W2, b2, W3, b3):
    src = edge_index[0].astype(jnp.int32)
    dst = edge_index[1].astype(jnp.int32)
    npad_e = EPAD - E
    srcp = jnp.concatenate([src, jnp.zeros((npad_e,), jnp.int32)])
    dstp = jnp.concatenate(
        [dst, N + (jnp.arange(npad_e, dtype=jnp.int32) % (NPAD - N))])
    xp = jnp.pad(x, ((0, NPAD - N), (0, 0)))

    dp = _sc_count(dstp)
    dis, y1l, y1r = _tc_dis_y(dp[0, :, 0:1], dp[1, :, 0:1], xp, W1)

    P1 = _sc_agg[64](y1l, y1r, srcp, dstp)
    y2l, y2r = _tc_mid(P1[0], P1[1], y1l, y1r, dis, b1.reshape(1, -1), W2)

    P2 = _sc_agg[32](y2l, y2r, srcp, dstp)
    w3p = jnp.pad(W3, ((0, 0), (0, 24)))
    y3l, y3r = _tc_mid(P2[0], P2[1], y2l, y2r, dis, b2.reshape(1, -1), w3p)

    P3 = _sc_agg[32](y3l, y3r, srcp, dstp)
    out = _tc_fin(P3[0], P3[1], y3l, y3r, dis,
                  jnp.pad(b3, (0, 24)).reshape(1, -1))
    return out[:N, :40]


# speedup vs baseline: 1.0212x; 1.0212x over previous
import functools

import jax
import jax.numpy as jnp
from jax import lax
from jax.experimental import pallas as pl
from jax.experimental.pallas import tpu as pltpu
from jax.experimental.pallas import tpu_sc as plsc

N = 10000
NPAD = 10240
E = 320000
NC = 2
NS = 16
NW = NC * NS
K = 128
EPB = 2 * (-(-E // (NW * K * 2)))
NBLK = NW * EPB
BPT = NBLK // NS
EPAD = NBLK * K
RPT = NPAD // NS
ZR = 64
RB = 640
F32 = jnp.float32


def _make_sc_agg(d2):
    mesh = plsc.VectorSubcoreMesh(core_axis_name="c", subcore_axis_name="s")
    scratch = [
        pltpu.VMEM((2, K), jnp.int32),
        pltpu.VMEM((2, K), jnp.int32),
        pltpu.VMEM((2, K, d2), F32),
        pltpu.VMEM((ZR, d2), F32),
        pltpu.VMEM_SHARED((NPAD, d2), F32),
        pltpu.VMEM_SHARED((NPAD, d2), F32),
    ] + [pltpu.SemaphoreType.DMA] * 8

    @functools.partial(
        pl.kernel,
        out_type=jax.ShapeDtypeStruct((NC, NPAD, d2), F32),
        mesh=mesh,
        scratch_types=scratch,
        compiler_params=pltpu.CompilerParams(use_tc_tiling_on_sc=False),
    )
    def body(yl_hbm, yr_hbm, src_hbm, dst_hbm, out_hbm, sidx, didx, rows,
             zbuf, yloc, acc, sf0, sf1, sd0, sd1, sg0, sg1, ss0, ss1):
        s_src, s_dst = (sf0, sf1), (sd0, sd1)
        s_g, s_s = (sg0, sg1), (ss0, ss1)
        c = lax.axis_index("c")
        s = lax.axis_index("s")

        fill = jnp.zeros((16,), F32)

        @pl.loop(0, ZR)
        def _(i):
            @pl.loop(0, d2 // 16)
            def _(j):
                zbuf[i, pl.ds(j * 16, 16)] = fill

        @pl.loop(0, RPT // ZR)
        def _(t):
            pltpu.sync_copy(zbuf, acc.at[pl.ds(s * RPT + t * ZR, ZR)])

        @pl.when(c == 0)
        def _():
            pltpu.sync_copy(yl_hbm.at[pl.ds(s * RPT, RPT)],
                            yloc.at[pl.ds(s * RPT, RPT)])

        @pl.when(c == 1)
        def _():
            pltpu.sync_copy(yr_hbm.at[pl.ds(s * RPT, RPT)],
                            yloc.at[pl.ds(s * RPT, RPT)])

        plsc.subcore_barrier()

        base0 = s * (BPT * K)

        def fetch(t, p):
            off = base0 + t * K
            pltpu.async_copy(src_hbm.at[pl.ds(off, K)], sidx.at[p], s_src[p])
            pltpu.async_copy(dst_hbm.at[pl.ds(off, K)], didx.at[p], s_dst[p])

        def wait_fetch(p):
            pltpu.make_async_copy(src_hbm.at[pl.ds(0, K)], sidx.at[p],
                                  s_src[p]).wait()
            pltpu.make_async_copy(dst_hbm.at[pl.ds(0, K)], didx.at[p],
                                  s_dst[p]).wait()

        def do_gather(p):
            pltpu.async_copy(yloc.at[sidx.at[p]], rows.at[p], s_g[p])

        def wait_gather(p):
            pltpu.make_async_copy(yloc.at[sidx.at[p]], rows.at[p],
                                  s_g[p]).wait()

        def do_scatter(p):
            pltpu.async_copy(rows.at[p], acc.at[didx.at[p]], s_s[p], add=True)

        def wait_scatter(p):
            pltpu.make_async_copy(rows.at[p], acc.at[didx.at[p]],
                                  s_s[p]).wait()

        fetch(0, 0)
        fetch(1, 1)
        wait_fetch(0)
        do_gather(0)

        @pl.loop(0, (BPT - 2) // 2)
        def _(it):
            t0 = it * 2
            for p in (0, 1):
                wait_gather(p)
                do_scatter(p)
                wait_fetch(1 - p)
                do_gather(1 - p)
                wait_scatter(p)
                fetch(t0 + p + 2, p)

        wait_gather(0)
        do_scatter(0)
        wait_fetch(1)
        do_gather(1)
        wait_scatter(0)
        wait_gather(1)
        do_scatter(1)
        wait_scatter(1)

        plsc.subcore_barrier()
        pltpu.sync_copy(acc.at[pl.ds(s * RPT, RPT)],
                        out_hbm.at[c, pl.ds(s * RPT, RPT)])

    return body


def _make_sc_count(d):
    mesh = plsc.VectorSubcoreMesh(core_axis_name="c", subcore_axis_name="s")
    scratch = [
        pltpu.VMEM((2, K), jnp.int32),
        pltpu.VMEM((K, d), F32),
        pltpu.VMEM((ZR, d), F32),
        pltpu.VMEM_SHARED((NPAD, d), F32),
    ] + [pltpu.SemaphoreType.DMA] * 4

    @functools.partial(
        pl.kernel,
        out_type=jax.ShapeDtypeStruct((NC, NPAD, d), F32),
        mesh=mesh,
        scratch_types=scratch,
        compiler_params=pltpu.CompilerParams(use_tc_tiling_on_sc=False),
    )
    def body(dst_hbm, out_hbm, didx, rows, zbuf, acc, sd0, sd1, ss0, ss1):
        s_dst, s_s = (sd0, sd1), (ss0, ss1)
        c = lax.axis_index("c")
        s = lax.axis_index("s")
        wid = s * NC + c
        base0 = wid * (EPB * K)

        fill = jnp.zeros((16,), F32)
        ones = jnp.ones((16,), F32)

        @pl.loop(0, ZR)
        def _(i):
            @pl.loop(0, d // 16)
            def _(j):
                zbuf[i, pl.ds(j * 16, 16)] = fill

        @pl.loop(0, K)
        def _(i):
            @pl.loop(0, d // 16)
            def _(j):
                rows[i, pl.ds(j * 16, 16)] = ones

        @pl.loop(0, RPT // ZR)
        def _(t):
            pltpu.sync_copy(zbuf, acc.at[pl.ds(s * RPT + t * ZR, ZR)])

        plsc.subcore_barrier()

        def fetch(t, p):
            off = base0 + t * K
            pltpu.async_copy(dst_hbm.at[pl.ds(off, K)], didx.at[p], s_dst[p])

        def wait_fetch(p):
            pltpu.make_async_copy(dst_hbm.at[pl.ds(0, K)], didx.at[p],
                                  s_dst[p]).wait()

        def do_scatter(p):
            pltpu.async_copy(rows, acc.at[didx.at[p]], s_s[p], add=True)

        def wait_scatter(p):
            pltpu.make_async_copy(rows, acc.at[didx.at[p]], s_s[p]).wait()

        fetch(0, 0)
        fetch(1, 1)

        @pl.loop(0, (EPB - 2) // 2)
        def _(it):
            t0 = it * 2
            for p in (0, 1):
                wait_fetch(p)
                do_scatter(p)
                wait_scatter(p)
                fetch(t0 + p + 2, p)

        for p in (0, 1):
            wait_fetch(p)
            do_scatter(p)
            wait_scatter(p)

        plsc.subcore_barrier()
        pltpu.sync_copy(acc.at[pl.ds(s * RPT, RPT)],
                        out_hbm.at[c, pl.ds(s * RPT, RPT)])

    return body


_sc_count = _make_sc_count(16)
_sc_agg = {64: _make_sc_agg(64), 32: _make_sc_agg(32)}


def _dis_y_body(d0, d1, x, w, dis_o, yl_o, yr_o):
    deg = d0[...] + d1[...] + 1.0
    dis = lax.rsqrt(deg)
    dis_o[...] = dis
    y = dis * jnp.dot(x[...], w[...], preferred_element_type=F32)
    d2 = y.shape[1] // 2
    yl_o[...] = y[:, :d2]
    yr_o[...] = y[:, d2:]


def _mid_body(p0, p1, yl, yr, dis, b, w, yl_o, yr_o):
    h = dis[...] * jnp.concatenate(
        [p0[...] + yl[...], p1[...] + yr[...]], axis=1) + b[...]
    h = jnp.maximum(h, 0.0)
    y = dis[...] * jnp.dot(h, w[...], preferred_element_type=F32)
    d2 = y.shape[1] // 2
    yl_o[...] = y[:, :d2]
    yr_o[...] = y[:, d2:]


def _fin_body(p0, p1, yl, yr, dis, b, out):
    out[...] = dis[...] * jnp.concatenate(
        [p0[...] + yl[...], p1[...] + yr[...]], axis=1) + b[...]


def _col(i):
    return (i, 0)


def _rep(i):
    return (0, 0)


def _tc_dis_y(dp0, dp1, xp, w):
    din, dout = w.shape
    d2 = dout // 2
    return pl.pallas_call(
        _dis_y_body,
        grid=(NPAD // RB,),
        in_specs=[
            pl.BlockSpec((RB, 1), _col),
            pl.BlockSpec((RB, 1), _col),
            pl.BlockSpec((RB, din), _col),
            pl.BlockSpec((din, dout), _rep),
        ],
        out_specs=[
            pl.BlockSpec((RB, 1), _col),
            pl.BlockSpec((RB, d2), _col),
            pl.BlockSpec((RB, d2), _col),
        ],
        out_shape=[
            jax.ShapeDtypeStruct((NPAD, 1), F32),
            jax.ShapeDtypeStruct((NPAD, d2), F32),
            jax.ShapeDtypeStruct((NPAD, d2), F32),
        ],
    )(dp0, dp1, xp, w)


def _tc_mid(p0, p1, yl, yr, dis, b, w):
    din, dout = w.shape
    d2i, d2o = din // 2, dout // 2
    return pl.pallas_call(
        _mid_body,
        grid=(NPAD // RB,),
        in_specs=[
            pl.BlockSpec((RB, d2i), _col),
            pl.BlockSpec((RB, d2i), _col),
            pl.BlockSpec((RB, d2i), _col),
            pl.BlockSpec((RB, d2i), _col),
            pl.BlockSpec((RB, 1), _col),
            pl.BlockSpec((1, din), _rep),
            pl.BlockSpec((din, dout), _rep),
        ],
        out_specs=[
            pl.BlockSpec((RB, d2o), _col),
            pl.BlockSpec((RB, d2o), _col),
        ],
        out_shape=[
            jax.ShapeDtypeStruct((NPAD, d2o), F32),
            jax.ShapeDtypeStruct((NPAD, d2o), F32),
        ],
    )(p0, p1, yl, yr, dis, b, w)


def _tc_fin(p0, p1, yl, yr, dis, b):
    d2 = p0.shape[1]
    return pl.pallas_call(
        _fin_body,
        grid=(NPAD // RB,),
        in_specs=[
            pl.BlockSpec((RB, d2), _col),
            pl.BlockSpec((RB, d2), _col),
            pl.BlockSpec((RB, d2), _col),
            pl.BlockSpec((RB, d2), _col),
            pl.BlockSpec((RB, 1), _col),
            pl.BlockSpec((1, 2 * d2), _rep),
        ],
        out_specs=pl.BlockSpec((RB, 2 * d2), _col),
        out_shape=jax.ShapeDtypeStruct((NPAD, 2 * d2), F32),
    )(p0, p1, yl, yr, dis, b)


def kernel(x, edge_index, W1, b1, W2, b2, W3, b3):
    src = edge_index[0].astype(jnp.int32)
    dst = edge_index[1].astype(jnp.int32)
    npad_e = EPAD - E
    srcp = jnp.concatenate([src, jnp.zeros((npad_e,), jnp.int32)])
    dstp = jnp.concatenate(
        [dst, N + (jnp.arange(npad_e, dtype=jnp.int32) % (NPAD - N))])
    xp = jnp.pad(x, ((0, NPAD - N), (0, 0)))

    dp = _sc_count(dstp)
    dis, y1l, y1r = _tc_dis_y(dp[0, :, 0:1], dp[1, :, 0:1], xp, W1)

    P1 = _sc_agg[64](y1l, y1r, srcp, dstp)
    y2l, y2r = _tc_mid(P1[0], P1[1], y1l, y1r, dis, b1.reshape(1, -1), W2)

    P2 = _sc_agg[32](y2l, y2r, srcp, dstp)
    w3p = jnp.pad(W3, ((0, 0), (0, 24)))
    y3l, y3r = _tc_mid(P2[0], P2[1], y2l, y2r, dis, b2.reshape(1, -1), w3p)

    P3 = _sc_agg[32](y3l, y3r, srcp, dstp)
    out = _tc_fin(P3[0], P3[1], y3l, y3r, dis,
                  jnp.pad(b3, (0, 24)).reshape(1, -1))
    return out[:N, :40]


# speedup vs baseline: 1.1411x; 1.1174x over previous
import functools

import jax
import jax.numpy as jnp
from jax import lax
from jax.experimental import pallas as pl
from jax.experimental.pallas import tpu as pltpu
from jax.experimental.pallas import tpu_sc as plsc

N = 10000
NPAD = 10240
E = 320000
NC = 2
NS = 16
NW = NC * NS
K = 128
EPB = 2 * (-(-E // (NW * K * 2)))
NBLK = NW * EPB
BPT = NBLK // NS
EPAD = NBLK * K
RPT = NPAD // NS
ZR = 64
RB = 640
F32 = jnp.float32


def _make_sc_agg(d2):
    assert BPT >= 4 and (BPT - 4) % 3 == 0
    mesh = plsc.VectorSubcoreMesh(core_axis_name="c", subcore_axis_name="s")
    scratch = [
        pltpu.VMEM((3, K), jnp.int32),
        pltpu.VMEM((3, K), jnp.int32),
        pltpu.VMEM((3, K, d2), F32),
        pltpu.VMEM((ZR, d2), F32),
        pltpu.VMEM_SHARED((NPAD, d2), F32),
        pltpu.VMEM_SHARED((NPAD, d2), F32),
    ] + [pltpu.SemaphoreType.DMA] * 12

    @functools.partial(
        pl.kernel,
        out_type=jax.ShapeDtypeStruct((NC, NPAD, d2), F32),
        mesh=mesh,
        scratch_types=scratch,
        compiler_params=pltpu.CompilerParams(use_tc_tiling_on_sc=False),
    )
    def body(yl_hbm, yr_hbm, src_hbm, dst_hbm, out_hbm, sidx, didx, rows,
             zbuf, yloc, acc, sf0, sf1, sf2, sd0, sd1, sd2, sg0, sg1, sg2,
             ss0, ss1, ss2):
        s_src, s_dst = (sf0, sf1, sf2), (sd0, sd1, sd2)
        s_g, s_s = (sg0, sg1, sg2), (ss0, ss1, ss2)
        c = lax.axis_index("c")
        s = lax.axis_index("s")

        fill = jnp.zeros((16,), F32)

        @pl.loop(0, ZR)
        def _(i):
            @pl.loop(0, d2 // 16)
            def _(j):
                zbuf[i, pl.ds(j * 16, 16)] = fill

        @pl.loop(0, RPT // ZR)
        def _(t):
            pltpu.sync_copy(zbuf, acc.at[pl.ds(s * RPT + t * ZR, ZR)])

        @pl.when(c == 0)
        def _():
            pltpu.sync_copy(yl_hbm.at[pl.ds(s * RPT, RPT)],
                            yloc.at[pl.ds(s * RPT, RPT)])

        @pl.when(c == 1)
        def _():
            pltpu.sync_copy(yr_hbm.at[pl.ds(s * RPT, RPT)],
                            yloc.at[pl.ds(s * RPT, RPT)])

        plsc.subcore_barrier()

        base0 = s * (BPT * K)

        def fetch(t, p):
            off = base0 + t * K
            pltpu.async_copy(src_hbm.at[pl.ds(off, K)], sidx.at[p], s_src[p])
            pltpu.async_copy(dst_hbm.at[pl.ds(off, K)], didx.at[p], s_dst[p])

        def wait_fetch(p):
            pltpu.make_async_copy(src_hbm.at[pl.ds(0, K)], sidx.at[p],
                                  s_src[p]).wait()
            pltpu.make_async_copy(dst_hbm.at[pl.ds(0, K)], didx.at[p],
                                  s_dst[p]).wait()

        def do_gather(p):
            pltpu.async_copy(yloc.at[sidx.at[p]], rows.at[p], s_g[p])

        def wait_gather(p):
            pltpu.make_async_copy(yloc.at[sidx.at[p]], rows.at[p],
                                  s_g[p]).wait()

        def do_scatter(p):
            pltpu.async_copy(rows.at[p], acc.at[didx.at[p]], s_s[p], add=True)

        def wait_scatter(p):
            pltpu.make_async_copy(rows.at[p], acc.at[didx.at[p]],
                                  s_s[p]).wait()

        fetch(0, 0)
        fetch(1, 1)
        wait_fetch(0)
        do_gather(0)

        wait_gather(0)
        do_scatter(0)
        fetch(2, 2)
        wait_fetch(1)
        do_gather(1)
        wait_gather(1)
        do_scatter(1)
        wait_scatter(0)
        fetch(3, 0)
        wait_fetch(2)
        do_gather(2)

        @pl.loop(0, (BPT - 4) // 3)
        def _(it):
            t0 = 2 + it * 3
            for j in (0, 1, 2):
                p = (2 + j) % 3
                r = (p + 1) % 3
                q = (p + 2) % 3
                wait_gather(p)
                do_scatter(p)
                wait_scatter(q)
                fetch(t0 + j + 2, q)
                wait_fetch(r)
                do_gather(r)

        pe = (BPT - 2) % 3
        re = (pe + 1) % 3
        qe = (pe + 2) % 3
        wait_gather(pe)
        do_scatter(pe)
        wait_scatter(qe)
        wait_fetch(re)
        do_gather(re)
        wait_gather(re)
        do_scatter(re)
        wait_scatter(pe)
        wait_scatter(re)

        plsc.subcore_barrier()
        pltpu.sync_copy(acc.at[pl.ds(s * RPT, RPT)],
                        out_hbm.at[c, pl.ds(s * RPT, RPT)])

    return body


def _make_sc_count(d):
    mesh = plsc.VectorSubcoreMesh(core_axis_name="c", subcore_axis_name="s")
    scratch = [
        pltpu.VMEM((2, K), jnp.int32),
        pltpu.VMEM((K, d), F32),
        pltpu.VMEM((ZR, d), F32),
        pltpu.VMEM_SHARED((NPAD, d), F32),
    ] + [pltpu.SemaphoreType.DMA] * 4

    @functools.partial(
        pl.kernel,
        out_type=jax.ShapeDtypeStruct((NC, NPAD, d), F32),
        mesh=mesh,
        scratch_types=scratch,
        compiler_params=pltpu.CompilerParams(use_tc_tiling_on_sc=False),
    )
    def body(dst_hbm, out_hbm, didx, rows, zbuf, acc, sd0, sd1, ss0, ss1):
        s_dst, s_s = (sd0, sd1), (ss0, ss1)
        c = lax.axis_index("c")
        s = lax.axis_index("s")
        wid = s * NC + c
        base0 = wid * (EPB * K)

        fill = jnp.zeros((16,), F32)
        ones = jnp.ones((16,), F32)

        @pl.loop(0, ZR)
        def _(i):
            @pl.loop(0, d // 16)
            def _(j):
                zbuf[i, pl.ds(j * 16, 16)] = fill

        @pl.loop(0, K)
        def _(i):
            @pl.loop(0, d // 16)
            def _(j):
                rows[i, pl.ds(j * 16, 16)] = ones

        @pl.loop(0, RPT // ZR)
        def _(t):
            pltpu.sync_copy(zbuf, acc.at[pl.ds(s * RPT + t * ZR, ZR)])

        plsc.subcore_barrier()

        def fetch(t, p):
            off = base0 + t * K
            pltpu.async_copy(dst_hbm.at[pl.ds(off, K)], didx.at[p], s_dst[p])

        def wait_fetch(p):
            pltpu.make_async_copy(dst_hbm.at[pl.ds(0, K)], didx.at[p],
                                  s_dst[p]).wait()

        def do_scatter(p):
            pltpu.async_copy(rows, acc.at[didx.at[p]], s_s[p], add=True)

        def wait_scatter(p):
            pltpu.make_async_copy(rows, acc.at[didx.at[p]], s_s[p]).wait()

        fetch(0, 0)
        fetch(1, 1)

        @pl.loop(0, (EPB - 2) // 2)
        def _(it):
            t0 = it * 2
            for p in (0, 1):
                wait_fetch(p)
                do_scatter(p)
                wait_scatter(p)
                fetch(t0 + p + 2, p)

        for p in (0, 1):
            wait_fetch(p)
            do_scatter(p)
            wait_scatter(p)

        plsc.subcore_barrier()
        pltpu.sync_copy(acc.at[pl.ds(s * RPT, RPT)],
                        out_hbm.at[c, pl.ds(s * RPT, RPT)])

    return body


_sc_count = _make_sc_count(16)
_sc_agg = {64: _make_sc_agg(64), 32: _make_sc_agg(32)}


def _dis_y_body(d0, d1, x, w, dis_o, yl_o, yr_o):
    deg = d0[...] + d1[...] + 1.0
    dis = lax.rsqrt(deg)
    dis_o[...] = dis
    y = dis * jnp.dot(x[...], w[...], preferred_element_type=F32)
    d2 = y.shape[1] // 2
    yl_o[...] = y[:, :d2]
    yr_o[...] = y[:, d2:]


def _mid_body(p0, p1, yl, yr, dis, b, w, yl_o, yr_o):
    h = dis[...] * jnp.concatenate(
        [p0[...] + yl[...], p1[...] + yr[...]], axis=1) + b[...]
    h = jnp.maximum(h, 0.0)
    y = dis[...] * jnp.dot(h, w[...], preferred_element_type=F32)
    d2 = y.shape[1] // 2
    yl_o[...] = y[:, :d2]
    yr_o[...] = y[:, d2:]


def _fin_body(p0, p1, yl, yr, dis, b, out):
    out[...] = dis[...] * jnp.concatenate(
        [p0[...] + yl[...], p1[...] + yr[...]], axis=1) + b[...]


def _col(i):
    return (i, 0)


def _rep(i):
    return (0, 0)


def _tc_dis_y(dp0, dp1, xp, w):
    din, dout = w.shape
    d2 = dout // 2
    return pl.pallas_call(
        _dis_y_body,
        grid=(NPAD // RB,),
        in_specs=[
            pl.BlockSpec((RB, 1), _col),
            pl.BlockSpec((RB, 1), _col),
            pl.BlockSpec((RB, din), _col),
            pl.BlockSpec((din, dout), _rep),
        ],
        out_specs=[
            pl.BlockSpec((RB, 1), _col),
            pl.BlockSpec((RB, d2), _col),
            pl.BlockSpec((RB, d2), _col),
        ],
        out_shape=[
            jax.ShapeDtypeStruct((NPAD, 1), F32),
            jax.ShapeDtypeStruct((NPAD, d2), F32),
            jax.ShapeDtypeStruct((NPAD, d2), F32),
        ],
    )(dp0, dp1, xp, w)


def _tc_mid(p0, p1, yl, yr, dis, b, w):
    din, dout = w.shape
    d2i, d2o = din // 2, dout // 2
    return pl.pallas_call(
        _mid_body,
        grid=(NPAD // RB,),
        in_specs=[
            pl.BlockSpec((RB, d2i), _col),
            pl.BlockSpec((RB, d2i), _col),
            pl.BlockSpec((RB, d2i), _col),
            pl.BlockSpec((RB, d2i), _col),
            pl.BlockSpec((RB, 1), _col),
            pl.BlockSpec((1, din), _rep),
            pl.BlockSpec((din, dout), _rep),
        ],
        out_specs=[
            pl.BlockSpec((RB, d2o), _col),
            pl.BlockSpec((RB, d2o), _col),
        ],
        out_shape=[
            jax.ShapeDtypeStruct((NPAD, d2o), F32),
            jax.ShapeDtypeStruct((NPAD, d2o), F32),
        ],
    )(p0, p1, yl, yr, dis, b, w)


def _tc_fin(p0, p1, yl, yr, dis, b):
    d2 = p0.shape[1]
    return pl.pallas_call(
        _fin_body,
        grid=(NPAD // RB,),
        in_specs=[
            pl.BlockSpec((RB, d2), _col),
            pl.BlockSpec((RB, d2), _col),
            pl.BlockSpec((RB, d2), _col),
            pl.BlockSpec((RB, d2), _col),
            pl.BlockSpec((RB, 1), _col),
            pl.BlockSpec((1, 2 * d2), _rep),
        ],
        out_specs=pl.BlockSpec((RB, 2 * d2), _col),
        out_shape=jax.ShapeDtypeStruct((NPAD, 2 * d2), F32),
    )(p0, p1, yl, yr, dis, b)


def kernel(x, edge_index, W1, b1, W2, b2, W3, b3):
    src = edge_index[0].astype(jnp.int32)
    dst = edge_index[1].astype(jnp.int32)
    npad_e = EPAD - E
    srcp = jnp.concatenate([src, jnp.zeros((npad_e,), jnp.int32)])
    dstp = jnp.concatenate(
        [dst, N + (jnp.arange(npad_e, dtype=jnp.int32) % (NPAD - N))])
    xp = jnp.pad(x, ((0, NPAD - N), (0, 0)))

    dp = _sc_count(dstp)
    dis, y1l, y1r = _tc_dis_y(dp[0, :, 0:1], dp[1, :, 0:1], xp, W1)

    P1 = _sc_agg[64](y1l, y1r, srcp, dstp)
    y2l, y2r = _tc_mid(P1[0], P1[1], y1l, y1r, dis, b1.reshape(1, -1), W2)

    P2 = _sc_agg[32](y2l, y2r, srcp, dstp)
    w3p = jnp.pad(W3, ((0, 0), (0, 24)))
    y3l, y3r = _tc_mid(P2[0], P2[1], y2l, y2r, dis, b2.reshape(1, -1), w3p)

    P3 = _sc_agg[32](y3l, y3r, srcp, dstp)
    out = _tc_fin(P3[0], P3[1], y3l, y3r, dis,
                  jnp.pad(b3, (0, 24)).reshape(1, -1))
    return out[:N, :40]
